# baseline (speedup 1.0000x reference)
import functools

import jax
import jax.numpy as jnp
from jax.experimental import pallas as pl
from jax.experimental.pallas import tpu as pltpu

T = 2048
H = 2048
F = 7168
E = 8
K = 2

B = 512
N = T * K
NB = N // B + (E - 1)
NPAD = NB * B
FT = 512
NF = F // FT


def _dispatch_meta(selected_experts, routing_weights):
    e_flat = selected_experts.reshape(-1).astype(jnp.int32)
    rw_flat = routing_weights.reshape(-1)
    counts = jnp.bincount(e_flat, length=E)
    nb = (counts + B - 1) // B
    gsz = nb * B
    start = jnp.concatenate([jnp.zeros(1, jnp.int32),
                             jnp.cumsum(gsz)[:-1].astype(jnp.int32)])
    seg0 = jnp.concatenate([jnp.zeros(1, jnp.int32),
                            jnp.cumsum(counts)[:-1].astype(jnp.int32)])
    order = jnp.argsort(e_flat, stable=True)
    e_sorted = e_flat[order]
    ranks = jnp.arange(N, dtype=jnp.int32) - seg0[e_sorted]
    pos_sorted = start[e_sorted] + ranks
    pos_flat = jnp.zeros(N, jnp.int32).at[order].set(pos_sorted)
    tok_s = jnp.zeros(NPAD, jnp.int32).at[pos_sorted].set(
        (order // K).astype(jnp.int32))
    rw_s = jnp.zeros(NPAD, jnp.float32).at[pos_sorted].set(rw_flat[order])

    nb_cum = jnp.cumsum(nb).astype(jnp.int32)
    total_blocks = nb_cum[-1]
    bidx = jnp.arange(NB, dtype=jnp.int32)
    blk_expert = jnp.searchsorted(nb_cum, bidx, side="right").astype(jnp.int32)
    valid = (bidx < total_blocks).astype(jnp.int32)
    last_b = jnp.maximum(total_blocks - 1, 0)
    last_e = blk_expert[last_b]
    emap = jnp.where(valid == 1, jnp.minimum(blk_expert, E - 1), last_e)
    bmap = jnp.where(valid == 1, bidx, last_b)
    return tok_s, rw_s, pos_flat, bmap, emap, valid


def _mlp_body(bmap_ref, emap_ref, valid_ref,
              xs_ref, w1_ref, w2_ref, w3_ref, rw_ref, out_ref):
    b = pl.program_id(0)
    f = pl.program_id(1)
    is_valid = valid_ref[b] == 1

    @pl.when(f == 0)
    def _():
        out_ref[...] = jnp.zeros_like(out_ref)

    @pl.when(is_valid)
    def _():
        x = xs_ref[...]
        a = jnp.dot(x, w1_ref[0], preferred_element_type=jnp.float32)
        c = jnp.dot(x, w2_ref[0], preferred_element_type=jnp.float32)
        p = jax.nn.silu(a) * c * rw_ref[...]
        out_ref[...] += jnp.dot(p, w3_ref[0],
                                preferred_element_type=jnp.float32)


def _grouped_mlp(xs, rw_s, bmap, emap, valid, w1, w2, w3):
    def xs_map(b, f, bmap, emap, valid):
        return bmap[b], 0

    def w12_map(b, f, bmap, emap, valid):
        fi = jnp.where(valid[b] == 1, f, NF - 1)
        return emap[b], 0, fi

    def w3_map(b, f, bmap, emap, valid):
        fi = jnp.where(valid[b] == 1, f, NF - 1)
        return emap[b], fi, 0

    def rw_map(b, f, bmap, emap, valid):
        return bmap[b], 0

    def out_map(b, f, bmap, emap, valid):
        return b, 0

    grid_spec = pltpu.PrefetchScalarGridSpec(
        num_scalar_prefetch=3,
        grid=(NB, NF),
        in_specs=[
            pl.BlockSpec((B, H), xs_map),
            pl.BlockSpec((1, H, FT), w12_map),
            pl.BlockSpec((1, H, FT), w12_map),
            pl.BlockSpec((1, FT, H), w3_map),
            pl.BlockSpec((B, 1), rw_map),
        ],
        out_specs=pl.BlockSpec((B, H), out_map),
    )
    return pl.pallas_call(
        _mlp_body,
        grid_spec=grid_spec,
        out_shape=jax.ShapeDtypeStruct((NPAD, H), jnp.float32),
        compiler_params=pltpu.CompilerParams(
            dimension_semantics=("arbitrary", "arbitrary"),
        ),
    )(bmap, emap, valid, xs, w1, w2, w3, rw_s.reshape(NPAD, 1))


def kernel(hidden_states, selected_experts, routing_weights, w1, w2, w3):
    tok_s, rw_s, pos_flat, bmap, emap, valid = _dispatch_meta(
        selected_experts, routing_weights)
    xs = hidden_states[tok_s]
    ys = _grouped_mlp(xs, rw_s, bmap, emap, valid, w1, w2, w3)
    pos = pos_flat.reshape(T, K)
    out = ys[pos[:, 0]] + ys[pos[:, 1]]
    return out


# speedup vs baseline: 1.0018x; 1.0018x over previous
import functools

import jax
import jax.numpy as jnp
from jax.experimental import pallas as pl
from jax.experimental.pallas import tpu as pltpu

T = 2048
H = 2048
F = 7168
E = 8
K = 2

B = 512
N = T * K
NB = N // B + (E - 1)
NPAD = NB * B
FT = 512
NF = F // FT


def _dispatch_meta(selected_experts, routing_weights):
    e_flat = selected_experts.reshape(-1).astype(jnp.int32)
    rw_flat = routing_weights.reshape(-1)
    counts = jnp.bincount(e_flat, length=E)
    nb = (counts + B - 1) // B
    gsz = nb * B
    start = jnp.concatenate([jnp.zeros(1, jnp.int32),
                             jnp.cumsum(gsz)[:-1].astype(jnp.int32)])
    seg0 = jnp.concatenate([jnp.zeros(1, jnp.int32),
                            jnp.cumsum(counts)[:-1].astype(jnp.int32)])
    order = jnp.argsort(e_flat, stable=True)
    e_sorted = e_flat[order]
    ranks = jnp.arange(N, dtype=jnp.int32) - seg0[e_sorted]
    pos_sorted = start[e_sorted] + ranks
    pos_flat = jnp.zeros(N, jnp.int32).at[order].set(pos_sorted)
    tok_s = jnp.zeros(NPAD, jnp.int32).at[pos_sorted].set(
        (order // K).astype(jnp.int32))
    rw_s = jnp.zeros(NPAD, jnp.float32).at[pos_sorted].set(rw_flat[order])

    nb_cum = jnp.cumsum(nb).astype(jnp.int32)
    total_blocks = nb_cum[-1]
    bidx = jnp.arange(NB, dtype=jnp.int32)
    blk_expert = jnp.searchsorted(nb_cum, bidx, side="right").astype(jnp.int32)
    valid = (bidx < total_blocks).astype(jnp.int32)
    last_b = jnp.maximum(total_blocks - 1, 0)
    last_e = blk_expert[last_b]
    emap = jnp.where(valid == 1, jnp.minimum(blk_expert, E - 1), last_e)
    bmap = jnp.where(valid == 1, bidx, last_b)
    return tok_s, rw_s, pos_flat, bmap, emap, valid


def _mlp_body(bmap_ref, emap_ref, valid_ref,
              xs_ref, w1_ref, w2_ref, w3_ref, rw_ref, out_ref):
    b = pl.program_id(0)
    f = pl.program_id(1)
    is_valid = valid_ref[b] == 1

    @pl.when(f == 0)
    def _():
        out_ref[...] = jnp.zeros_like(out_ref)

    @pl.when(is_valid)
    def _():
        x = xs_ref[...].astype(jnp.bfloat16)
        a = jnp.dot(x, w1_ref[0].astype(jnp.bfloat16),
                    preferred_element_type=jnp.float32)
        c = jnp.dot(x, w2_ref[0].astype(jnp.bfloat16),
                    preferred_element_type=jnp.float32)
        p = (jax.nn.silu(a) * c * rw_ref[...]).astype(jnp.bfloat16)
        out_ref[...] += jnp.dot(p, w3_ref[0].astype(jnp.bfloat16),
                                preferred_element_type=jnp.float32)


def _grouped_mlp(xs, rw_s, bmap, emap, valid, w1, w2, w3):
    def xs_map(b, f, bmap, emap, valid):
        return bmap[b], 0

    def w12_map(b, f, bmap, emap, valid):
        fi = jnp.where(valid[b] == 1, f, NF - 1)
        return emap[b], 0, fi

    def w3_map(b, f, bmap, emap, valid):
        fi = jnp.where(valid[b] == 1, f, NF - 1)
        return emap[b], fi, 0

    def rw_map(b, f, bmap, emap, valid):
        return bmap[b], 0

    def out_map(b, f, bmap, emap, valid):
        return b, 0

    grid_spec = pltpu.PrefetchScalarGridSpec(
        num_scalar_prefetch=3,
        grid=(NB, NF),
        in_specs=[
            pl.BlockSpec((B, H), xs_map),
            pl.BlockSpec((1, H, FT), w12_map),
            pl.BlockSpec((1, H, FT), w12_map),
            pl.BlockSpec((1, FT, H), w3_map),
            pl.BlockSpec((B, 1), rw_map),
        ],
        out_specs=pl.BlockSpec((B, H), out_map),
    )
    return pl.pallas_call(
        _mlp_body,
        grid_spec=grid_spec,
        out_shape=jax.ShapeDtypeStruct((NPAD, H), jnp.float32),
        compiler_params=pltpu.CompilerParams(
            dimension_semantics=("arbitrary", "arbitrary"),
        ),
    )(bmap, emap, valid, xs, w1, w2, w3, rw_s.reshape(NPAD, 1))


def kernel(hidden_states, selected_experts, routing_weights, w1, w2, w3):
    tok_s, rw_s, pos_flat, bmap, emap, valid = _dispatch_meta(
        selected_experts, routing_weights)
    xs = hidden_states[tok_s]
    ys = _grouped_mlp(xs, rw_s, bmap, emap, valid, w1, w2, w3)
    pos = pos_flat.reshape(T, K)
    out = ys[pos[:, 0]] + ys[pos[:, 1]]
    return out


# speedup vs baseline: 1.3633x; 1.3608x over previous
import functools

import jax
import jax.numpy as jnp
from jax.experimental import pallas as pl
from jax.experimental.pallas import tpu as pltpu

T = 2048
H = 2048
F = 7168
E = 8
K = 2

B = 576
N = T * K
NB = N // B + (E - 1)
NPAD = NB * B
FT = 512
NF = F // FT


def _dispatch_meta(selected_experts, routing_weights):
    e_flat = selected_experts.reshape(-1).astype(jnp.int32)
    rw_flat = routing_weights.reshape(-1)
    counts = jnp.bincount(e_flat, length=E)
    nb = (counts + B - 1) // B
    gsz = nb * B
    start = jnp.concatenate([jnp.zeros(1, jnp.int32),
                             jnp.cumsum(gsz)[:-1].astype(jnp.int32)])
    seg0 = jnp.concatenate([jnp.zeros(1, jnp.int32),
                            jnp.cumsum(counts)[:-1].astype(jnp.int32)])
    order = jnp.argsort(e_flat, stable=True)
    e_sorted = e_flat[order]
    ranks = jnp.arange(N, dtype=jnp.int32) - seg0[e_sorted]
    pos_sorted = start[e_sorted] + ranks
    pos_flat = jnp.zeros(N, jnp.int32).at[order].set(pos_sorted)
    tok_s = jnp.zeros(NPAD, jnp.int32).at[pos_sorted].set(
        (order // K).astype(jnp.int32))
    rw_s = jnp.zeros(NPAD, jnp.float32).at[pos_sorted].set(rw_flat[order])

    nb_cum = jnp.cumsum(nb).astype(jnp.int32)
    total_blocks = nb_cum[-1]
    bidx = jnp.arange(NB, dtype=jnp.int32)
    blk_expert = jnp.searchsorted(nb_cum, bidx, side="right").astype(jnp.int32)
    valid = (bidx < total_blocks).astype(jnp.int32)
    last_b = jnp.maximum(total_blocks - 1, 0)
    last_e = blk_expert[last_b]
    emap = jnp.where(valid == 1, jnp.minimum(blk_expert, E - 1), last_e)
    bmap = jnp.where(valid == 1, bidx, last_b)
    return tok_s, rw_s, pos_flat, bmap, emap, valid


def _mlp_body(bmap_ref, emap_ref, valid_ref,
              xs_ref, w1_ref, w2_ref, w3_ref, rw_ref, out_ref):
    b = pl.program_id(0)
    f = pl.program_id(1)
    is_valid = valid_ref[b] == 1

    @pl.when(f == 0)
    def _():
        out_ref[...] = jnp.zeros_like(out_ref)

    @pl.when(is_valid)
    def _():
        x = xs_ref[...].astype(jnp.bfloat16)
        a = jnp.dot(x, w1_ref[0].astype(jnp.bfloat16),
                    preferred_element_type=jnp.float32)
        c = jnp.dot(x, w2_ref[0].astype(jnp.bfloat16),
                    preferred_element_type=jnp.float32)
        p = (jax.nn.silu(a) * c * rw_ref[...]).astype(jnp.bfloat16)
        out_ref[...] += jnp.dot(p, w3_ref[0].astype(jnp.bfloat16),
                                preferred_element_type=jnp.float32)


def _grouped_mlp(xs, rw_s, bmap, emap, valid, w1, w2, w3):
    def xs_map(b, f, bmap, emap, valid):
        return bmap[b], 0

    def w12_map(b, f, bmap, emap, valid):
        fi = jnp.where(valid[b] == 1, f, NF - 1)
        return emap[b], 0, fi

    def w3_map(b, f, bmap, emap, valid):
        fi = jnp.where(valid[b] == 1, f, NF - 1)
        return emap[b], fi, 0

    def rw_map(b, f, bmap, emap, valid):
        return bmap[b], 0

    def out_map(b, f, bmap, emap, valid):
        return b, 0

    grid_spec = pltpu.PrefetchScalarGridSpec(
        num_scalar_prefetch=3,
        grid=(NB, NF),
        in_specs=[
            pl.BlockSpec((B, H), xs_map),
            pl.BlockSpec((1, H, FT), w12_map),
            pl.BlockSpec((1, H, FT), w12_map),
            pl.BlockSpec((1, FT, H), w3_map),
            pl.BlockSpec((B, 1), rw_map),
        ],
        out_specs=pl.BlockSpec((B, H), out_map),
    )
    return pl.pallas_call(
        _mlp_body,
        grid_spec=grid_spec,
        out_shape=jax.ShapeDtypeStruct((NPAD, H), jnp.float32),
        compiler_params=pltpu.CompilerParams(
            dimension_semantics=("arbitrary", "arbitrary"),
        ),
    )(bmap, emap, valid, xs, w1, w2, w3, rw_s.reshape(NPAD, 1))


def kernel(hidden_states, selected_experts, routing_weights, w1, w2, w3):
    tok_s, rw_s, pos_flat, bmap, emap, valid = _dispatch_meta(
        selected_experts, routing_weights)
    xs = hidden_states[tok_s]
    ys = _grouped_mlp(xs, rw_s, bmap, emap, valid, w1, w2, w3)
    pos = pos_flat.reshape(T, K)
    out = ys[pos[:, 0]] + ys[pos[:, 1]]
    return out
